# baseline (speedup 1.0000x reference)
import jax
import jax.numpy as jnp
from jax import lax
from jax.experimental import pallas as pl
from jax.experimental.pallas import tpu as pltpu
from jax.experimental.pallas import tpu_sc as plsc

_VOCAB = 1000000
_EMBED = 64
_BATCH = 16384
_X_MAX = 100.0
_ALPHA = 0.75

_NC = 2
_NS = 16
_NW = _NC * _NS
_BPW = _BATCH // _NW
_L = 16

_LN2 = 0.6931471805599453
_LOG_XMAX = 4.605170185988092


def _vlog(v):
    bits = plsc.bitcast(v, jnp.int32)
    e = ((bits >> 23) & 0xFF) - 127
    m = plsc.bitcast((bits & 0x7FFFFF) | 0x3F800000, jnp.float32)
    big = m > 1.4142135623730951
    m = jnp.where(big, m * 0.5, m)
    ef = (e + jnp.where(big, 1, 0)).astype(jnp.float32)
    z = (m - 1.0) / (m + 1.0)
    z2 = z * z
    s = 1.0 + z2 * ((1.0 / 3.0) + z2 * ((1.0 / 5.0) + z2 * ((1.0 / 7.0) + z2 * (1.0 / 9.0))))
    return ef * _LN2 + 2.0 * z * s


def _glove_sc_body(i_hbm, j_hbm, x_hbm, w_hbm, wt_hbm, b_hbm, bt_hbm,
                   part_hbm,
                   idx_i, idx_j, wi, wj, bi, bj, xs, acc_v, sem):
    wid = lax.axis_index("s") * _NC + lax.axis_index("c")
    base = wid * _BPW
    pltpu.sync_copy(i_hbm.at[pl.ds(base, _BPW)], idx_i)
    pltpu.sync_copy(j_hbm.at[pl.ds(base, _BPW)], idx_j)
    pltpu.sync_copy(x_hbm.at[pl.ds(base, _BPW)], xs)
    c1 = pltpu.async_copy(w_hbm.at[idx_i], wi, sem)
    c2 = pltpu.async_copy(wt_hbm.at[idx_j], wj, sem)
    c3 = pltpu.async_copy(b_hbm.at[idx_i], bi, sem)
    c4 = pltpu.async_copy(bt_hbm.at[idx_j], bj, sem)
    c1.wait()
    c2.wait()
    c3.wait()
    c4.wait()

    iot = lax.iota(jnp.int32, 16)
    zero16 = jnp.zeros((16,), jnp.int32)

    def blk(t, acc):
        r0 = t * _L
        rid = r0 + iot
        dots = jnp.zeros((_L,), jnp.float32)
        for d in range(_EMBED):
            dv = jnp.full((_L,), d, jnp.int32)
            vi = plsc.load_gather(wi, [rid, dv])
            vj = plsc.load_gather(wj, [rid, dv])
            dots = dots + vi * vj
        biv = plsc.load_gather(bi, [rid, zero16])
        bjv = plsc.load_gather(bj, [rid, zero16])
        xv = xs[pl.ds(r0, _L)]
        lx = _vlog(xv)
        wf = jnp.minimum(jnp.exp(_ALPHA * (lx - _LOG_XMAX)), 1.0)
        diff = dots + biv + bjv - lx
        return acc + wf * diff * diff

    acc = lax.fori_loop(0, _BPW // _L, blk, jnp.zeros((_L,), jnp.float32))
    acc_v[...] = acc
    pltpu.sync_copy(acc_v, part_hbm.at[wid])


def _finish_body(p_ref, o_ref):
    o_ref[...] = jnp.reshape(jnp.sum(p_ref[...]) * (1.0 / _BATCH), (1, 1))


@jax.jit
def _glove(i, j, x, w, wt, b, bt):
    mesh = plsc.VectorSubcoreMesh(core_axis_name="c", subcore_axis_name="s")
    kern = pl.kernel(
        _glove_sc_body,
        out_type=jax.ShapeDtypeStruct((_NW, _L), jnp.float32),
        mesh=mesh,
        compiler_params=pltpu.CompilerParams(
            needs_layout_passes=False, use_tc_tiling_on_sc=False),
        scratch_types=[
            pltpu.VMEM((_BPW,), jnp.int32),
            pltpu.VMEM((_BPW,), jnp.int32),
            pltpu.VMEM((_BPW, _EMBED), jnp.float32),
            pltpu.VMEM((_BPW, _EMBED), jnp.float32),
            pltpu.VMEM((_BPW, 1), jnp.float32),
            pltpu.VMEM((_BPW, 1), jnp.float32),
            pltpu.VMEM((_BPW,), jnp.float32),
            pltpu.VMEM((_L,), jnp.float32),
            pltpu.SemaphoreType.DMA,
        ],
    )
    parts = kern(i, j, x, w, wt,
                 b.reshape(_VOCAB, 1), bt.reshape(_VOCAB, 1))
    out = pl.pallas_call(
        _finish_body,
        out_shape=jax.ShapeDtypeStruct((1, 1), jnp.float32),
    )(parts)
    return out[0, 0]


def kernel(i, j, x, W, W_tilde, b, b_tilde):
    return _glove(i.astype(jnp.int32), j.astype(jnp.int32), x,
                  W, W_tilde, b, b_tilde)
